# baseline (speedup 1.0000x reference)
import functools

import jax
import jax.numpy as jnp
from jax import lax
from jax.experimental import pallas as pl
from jax.experimental.pallas import tpu as pltpu
from jax.experimental.pallas import tpu_sc as plsc

E = 8
H = 768
T = 2048
TM = 256
NTILES = T // TM
G = NTILES + E - 1
NW = 32
CHUNK = T // NW
LANES = 16


def _router_body(gate_t_ref, dest_ref, prob_ref, offs_ref, ranks_ref):
    gate_t = gate_t_ref[...]
    mx = jnp.max(gate_t, axis=0, keepdims=True)
    s = jnp.sum(jnp.exp(gate_t - mx), axis=0, keepdims=True)
    prob_ref[...] = 1.0 / s

    ioe = lax.broadcasted_iota(jnp.int32, (E, T), 0)
    idx = jnp.min(jnp.where(gate_t == mx, ioe, E), axis=0, keepdims=True)
    oh = (ioe == idx).astype(jnp.float32)

    iu0 = lax.broadcasted_iota(jnp.int32, (128, 128), 0)
    iu1 = lax.broadcasted_iota(jnp.int32, (128, 128), 1)
    upper = (iu0 <= iu1).astype(jnp.float32)
    carry = jnp.zeros((E, 1), jnp.float32)
    for i in range(T // 128):
        blk = oh[:, i * 128:(i + 1) * 128]
        c = jnp.dot(blk, upper, preferred_element_type=jnp.float32) + carry
        ranks_ref[:, i * 128:(i + 1) * 128] = c
        carry = c[:, 127:128]
    counts = carry

    il0 = lax.broadcasted_iota(jnp.int32, (E, E), 0)
    il1 = lax.broadcasted_iota(jnp.int32, (E, E), 1)
    strict = (il0 > il1).astype(jnp.float32)
    offs = jnp.dot(strict, counts, preferred_element_type=jnp.float32,
                   precision=lax.Precision.HIGHEST)

    dest_f = jnp.sum(oh * (offs + ranks_ref[...] - 1.0), axis=0, keepdims=True)
    dest_ref[...] = dest_f.astype(jnp.int32)
    offs9 = jnp.concatenate([jnp.zeros((1, 1), jnp.float32), offs + counts],
                            axis=0).astype(jnp.int32)
    offs_ref[...] = jnp.broadcast_to(offs9, (E + 1, 128))


_router = pl.pallas_call(
    _router_body,
    out_shape=[
        jax.ShapeDtypeStruct((1, T), jnp.int32),
        jax.ShapeDtypeStruct((1, T), jnp.float32),
        jax.ShapeDtypeStruct((E + 1, 128), jnp.int32),
    ],
    scratch_shapes=[pltpu.VMEM((E, T), jnp.float32)],
)


def _visit(g, off_ref):
    first, nv, cum = [], [], []
    run = 0
    for e in range(E):
        f = off_ref[e] // TM
        l = (off_ref[e + 1] - 1) // TM
        first.append(f)
        n = jnp.maximum(l - f + 1, 0)
        nv.append(n)
        run = run + n
        cum.append(run)
    eg = 0
    for e in range(E):
        eg = eg + (cum[e] <= g).astype(jnp.int32)
    eg = jnp.minimum(eg, E - 1)
    f_sel, x_sel = 0, 0
    for e in range(E):
        hit = (eg == e).astype(jnp.int32)
        f_sel = f_sel + hit * first[e]
        x_sel = x_sel + hit * (cum[e] - nv[e])
    tg = jnp.clip(f_sel + g - x_sel, 0, NTILES - 1)
    return tg, eg


def _gmm_body(off_ref, x_ref, w_ref, b_ref, o_ref):
    g = pl.program_id(0)
    m, e = _visit(g, off_ref)
    rows = m * TM + lax.broadcasted_iota(jnp.int32, (TM, 1), 0)
    mask = (rows >= off_ref[e]) & (rows < off_ref[e + 1])
    y = jnp.dot(x_ref[...], w_ref[0], preferred_element_type=jnp.float32)
    y = jnp.maximum(y + b_ref[0], 0.0)
    t_prev, _ = _visit(jnp.maximum(g - 1, 0), off_ref)
    first = jnp.logical_or(g == 0, m != t_prev)
    prev = jnp.where(first, 0.0, o_ref[...])
    o_ref[...] = jnp.where(mask, y, prev)


_gmm = pl.pallas_call(
    _gmm_body,
    grid_spec=pltpu.PrefetchScalarGridSpec(
        num_scalar_prefetch=1,
        grid=(G,),
        in_specs=[
            pl.BlockSpec((TM, H), lambda g, off: (_visit(g, off)[0], 0)),
            pl.BlockSpec((1, H, H), lambda g, off: (_visit(g, off)[1], 0, 0)),
            pl.BlockSpec((1, 1, H), lambda g, off: (_visit(g, off)[1], 0, 0)),
        ],
        out_specs=pl.BlockSpec((TM, H), lambda g, off: (_visit(g, off)[0], 0)),
    ),
    out_shape=jax.ShapeDtypeStruct((T, H), jnp.float32),
)


def _make_sc_kernels():
    mesh = plsc.VectorSubcoreMesh(core_axis_name="c", subcore_axis_name="s")

    @functools.partial(
        pl.kernel,
        mesh=mesh,
        out_type=jax.ShapeDtypeStruct((T, H), jnp.float32),
        scratch_types=[
            pltpu.VMEM((CHUNK,), jnp.int32),
            pltpu.VMEM((CHUNK, H), jnp.float32),
            pltpu.SemaphoreType.DMA,
        ],
    )
    def dispatch(x_hbm, dest_hbm, xs_hbm, idx_v, rows_v, sem):
        wid = lax.axis_index("s") * 2 + lax.axis_index("c")
        base = wid * CHUNK
        pltpu.sync_copy(dest_hbm.at[pl.ds(base, CHUNK)], idx_v)
        pltpu.sync_copy(x_hbm.at[pl.ds(base, CHUNK)], rows_v)
        pltpu.async_copy(rows_v, xs_hbm.at[idx_v], sem).wait()

    @functools.partial(
        pl.kernel,
        mesh=mesh,
        out_type=jax.ShapeDtypeStruct((T, H), jnp.float32),
        scratch_types=[
            pltpu.VMEM((CHUNK,), jnp.int32),
            pltpu.VMEM((CHUNK,), jnp.float32),
            pltpu.VMEM((CHUNK, H), jnp.float32),
            pltpu.SemaphoreType.DMA,
        ],
    )
    def combine(y_hbm, dest_hbm, prob_hbm, out_hbm, idx_v, p_v, rows_v, sem):
        wid = lax.axis_index("s") * 2 + lax.axis_index("c")
        base = wid * CHUNK
        pltpu.sync_copy(dest_hbm.at[pl.ds(base, CHUNK)], idx_v)
        pltpu.sync_copy(prob_hbm.at[pl.ds(base, CHUNK)], p_v)
        pltpu.async_copy(y_hbm.at[idx_v], rows_v, sem).wait()

        def scale_group(q, acc):
            pv = p_v[pl.ds(q * LANES, LANES)]
            for j in range(LANES):
                pr = jnp.broadcast_to(pv[j], (LANES,))
                r = q * LANES + j
                for c in range(H // LANES):
                    sl = pl.ds(c * LANES, LANES)
                    rows_v[r, sl] = rows_v[r, sl] * pr
            return acc

        lax.fori_loop(0, CHUNK // LANES, scale_group, 0)
        pltpu.sync_copy(rows_v, out_hbm.at[pl.ds(base, CHUNK)])

    return dispatch, combine


_make_sc_kernels = functools.cache(_make_sc_kernels)


def kernel(input, gate, We, be):
    dest2, prob2, offs2 = _router(gate.T)
    dest = dest2.reshape(T)
    prob = prob2.reshape(T)
    offs = offs2[:, 0]

    dispatch, combine = _make_sc_kernels()
    xs = dispatch(input, dest)
    ys = _gmm(offs, xs, We, be.reshape(E, 1, H))
    return combine(ys, dest, prob)


# speedup vs baseline: 1.0689x; 1.0689x over previous
import functools

import jax
import jax.numpy as jnp
from jax import lax
from jax.experimental import pallas as pl
from jax.experimental.pallas import tpu as pltpu
from jax.experimental.pallas import tpu_sc as plsc

E = 8
H = 768
T = 2048
TM = 256
NTILES = T // TM
G = NTILES + E - 1
NW = 32
CHUNK = T // NW
LANES = 16


def _router_body(gate_t_ref, dest_ref, prob_ref, offs_ref, ranks_ref):
    gate_t = gate_t_ref[...]
    mx = jnp.max(gate_t, axis=0, keepdims=True)
    s = jnp.sum(jnp.exp(gate_t - mx), axis=0, keepdims=True)
    prob_ref[...] = 1.0 / s

    ioe = lax.broadcasted_iota(jnp.int32, (E, T), 0)
    idx = jnp.min(jnp.where(gate_t == mx, ioe, E), axis=0, keepdims=True)
    oh = (ioe == idx).astype(jnp.float32)

    iu0 = lax.broadcasted_iota(jnp.int32, (128, 128), 0)
    iu1 = lax.broadcasted_iota(jnp.int32, (128, 128), 1)
    upper = (iu0 <= iu1).astype(jnp.float32)
    carry = jnp.zeros((E, 1), jnp.float32)
    for i in range(T // 128):
        blk = oh[:, i * 128:(i + 1) * 128]
        c = jnp.dot(blk, upper, preferred_element_type=jnp.float32) + carry
        ranks_ref[:, i * 128:(i + 1) * 128] = c
        carry = c[:, 127:128]
    counts = carry

    il0 = lax.broadcasted_iota(jnp.int32, (E, E), 0)
    il1 = lax.broadcasted_iota(jnp.int32, (E, E), 1)
    strict = (il0 > il1).astype(jnp.float32)
    offs = jnp.dot(strict, counts, preferred_element_type=jnp.float32,
                   precision=lax.Precision.HIGHEST)

    dest_f = jnp.sum(oh * (offs + ranks_ref[...] - 1.0), axis=0, keepdims=True)
    dest_ref[...] = dest_f.astype(jnp.int32)
    offs9 = jnp.concatenate([jnp.zeros((1, 1), jnp.float32), offs + counts],
                            axis=0).astype(jnp.int32)
    offs_ref[...] = jnp.broadcast_to(offs9, (E + 1, 128))


_router = pl.pallas_call(
    _router_body,
    out_shape=[
        jax.ShapeDtypeStruct((1, T), jnp.int32),
        jax.ShapeDtypeStruct((1, T), jnp.float32),
        jax.ShapeDtypeStruct((E + 1, 128), jnp.int32),
    ],
    scratch_shapes=[pltpu.VMEM((E, T), jnp.float32)],
)


def _gmm_body(tid_ref, eid_ref, off_ref, x_ref, w_ref, b_ref, o_ref):
    g = pl.program_id(0)
    m = tid_ref[g]
    e = eid_ref[g]
    rows = m * TM + lax.broadcasted_iota(jnp.int32, (TM, 1), 0)
    mask = (rows >= off_ref[e]) & (rows < off_ref[e + 1])
    y = jnp.dot(x_ref[...], w_ref[0], preferred_element_type=jnp.float32)
    y = jnp.maximum(y + b_ref[0], 0.0)
    t_prev = tid_ref[jnp.maximum(g - 1, 0)]
    first = jnp.logical_or(g == 0, m != t_prev)
    prev = jnp.where(first, 0.0, o_ref[...])
    o_ref[...] = jnp.where(mask, y, prev)


_gmm = pl.pallas_call(
    _gmm_body,
    grid_spec=pltpu.PrefetchScalarGridSpec(
        num_scalar_prefetch=3,
        grid=(G,),
        in_specs=[
            pl.BlockSpec((TM, H), lambda g, tid, eid, off: (tid[g], 0)),
            pl.BlockSpec((1, H, H), lambda g, tid, eid, off: (eid[g], 0, 0)),
            pl.BlockSpec((1, 1, H), lambda g, tid, eid, off: (eid[g], 0, 0)),
        ],
        out_specs=pl.BlockSpec((TM, H), lambda g, tid, eid, off: (tid[g], 0)),
    ),
    out_shape=jax.ShapeDtypeStruct((T, H), jnp.float32),
)


def _make_sc_kernels():
    mesh = plsc.VectorSubcoreMesh(core_axis_name="c", subcore_axis_name="s")

    @functools.partial(
        pl.kernel,
        mesh=mesh,
        out_type=jax.ShapeDtypeStruct((T, H), jnp.float32),
        scratch_types=[
            pltpu.VMEM((CHUNK,), jnp.int32),
            pltpu.VMEM((CHUNK, H), jnp.float32),
            pltpu.SemaphoreType.DMA,
        ],
    )
    def dispatch(x_hbm, dest_hbm, xs_hbm, idx_v, rows_v, sem):
        wid = lax.axis_index("s") * 2 + lax.axis_index("c")
        base = wid * CHUNK
        pltpu.sync_copy(dest_hbm.at[pl.ds(base, CHUNK)], idx_v)
        pltpu.sync_copy(x_hbm.at[pl.ds(base, CHUNK)], rows_v)
        pltpu.async_copy(rows_v, xs_hbm.at[idx_v], sem).wait()

    @functools.partial(
        pl.kernel,
        mesh=mesh,
        out_type=jax.ShapeDtypeStruct((T, H), jnp.float32),
        scratch_types=[
            pltpu.VMEM((CHUNK,), jnp.int32),
            pltpu.VMEM((CHUNK,), jnp.float32),
            pltpu.VMEM((CHUNK, H), jnp.float32),
            pltpu.SemaphoreType.DMA,
        ],
    )
    def combine(y_hbm, dest_hbm, prob_hbm, out_hbm, idx_v, p_v, rows_v, sem):
        wid = lax.axis_index("s") * 2 + lax.axis_index("c")
        base = wid * CHUNK
        pltpu.sync_copy(dest_hbm.at[pl.ds(base, CHUNK)], idx_v)
        pltpu.sync_copy(prob_hbm.at[pl.ds(base, CHUNK)], p_v)
        pltpu.async_copy(y_hbm.at[idx_v], rows_v, sem).wait()

        def scale_group(q, acc):
            pv = p_v[pl.ds(q * LANES, LANES)]
            for j in range(LANES):
                pr = jnp.broadcast_to(pv[j], (LANES,))
                r = q * LANES + j
                for c in range(H // LANES):
                    sl = pl.ds(c * LANES, LANES)
                    rows_v[r, sl] = rows_v[r, sl] * pr
            return acc

        lax.fori_loop(0, CHUNK // LANES, scale_group, 0)
        pltpu.sync_copy(rows_v, out_hbm.at[pl.ds(base, CHUNK)])

    return dispatch, combine


_make_sc_kernels = functools.cache(_make_sc_kernels)


def kernel(input, gate, We, be):
    dest2, prob2, offs2 = _router(gate.T)
    dest = dest2.reshape(T)
    prob = prob2.reshape(T)
    offs = offs2[:, 0]

    first = offs[:E] // TM
    last = (offs[1:] - 1) // TM
    nv = jnp.maximum(last - first + 1, 0)
    cum = jnp.cumsum(nv)
    gidx = jnp.arange(G, dtype=jnp.int32)
    e_g = jnp.minimum(
        jnp.sum((cum[None, :] <= gidx[:, None]).astype(jnp.int32), axis=1),
        E - 1)
    t_g = jnp.clip(first[e_g] + gidx - (cum - nv)[e_g], 0, NTILES - 1)

    dispatch, combine = _make_sc_kernels()
    xs = dispatch(input, dest)
    ys = _gmm(t_g, e_g, offs, xs, We, be.reshape(E, 1, H))
    return combine(ys, dest, prob)
